# Initial kernel scaffold; baseline (speedup 1.0000x reference)
import jax
import jax.numpy as jnp
from jax.experimental import pallas as pl


def kernel(er_src_H, er_dst_H, ee_src_H, ee_weight_H, er_src_T, er_dst_T, ee_src_T, ee_weight_T, embH, embT, WH_attn, bH_attn, wH_0, bH_0, WH_self, bH_self, WH_neigh, bH_neigh, WT_attn, bT_attn, wT_0, bT_0, WT_self, bT_self, WT_neigh, bT_neigh):
    raise NotImplementedError("write your pallas kernel here")


# speedup vs baseline: 5.6547x; 5.6547x over previous
import functools

import jax
import jax.numpy as jnp
from jax import lax
from jax.experimental import pallas as pl
from jax.experimental.pallas import tpu as pltpu
from jax.experimental.pallas import tpu_sc as plsc

NE = 50000
NR = 1000
N = NE + NR
D = 64
L = 2
NP = 51200
E_ER = 800000
E_EE = 200000
W = 128
ROWS = 6400
EROWS = 1600
EP = ROWS * W
EEP = EROWS * W
DSTLO = NE
DSTN = 1088
PADBASE = 51008
BD = 1024

_SC_MESH = plsc.VectorSubcoreMesh(core_axis_name="c", subcore_axis_name="s")
_STG = NP // 16


def _dense_body(x_ref, w_ref, b_ref, asrc_ref, adst_ref, m_ref, slf_ref):
    y = jnp.dot(x_ref[...], w_ref[...], preferred_element_type=jnp.float32)
    y = y + b_ref[...]
    asrc_ref[...] = jnp.stack([y[:, 0:32], y[:, 32:64]])
    adst_ref[...] = jnp.stack([y[:, 64:96], y[:, 96:128]])
    m_ref[...] = jnp.stack([y[:, 128:160], y[:, 160:192]])
    slf_ref[...] = y[:, 192:256]


_dense = pl.pallas_call(
    _dense_body,
    grid=(NP // BD,),
    in_specs=[
        pl.BlockSpec((BD, D), lambda i: (i, 0)),
        pl.BlockSpec((D, 4 * D), lambda i: (0, 0)),
        pl.BlockSpec((1, 4 * D), lambda i: (0, 0)),
    ],
    out_specs=[
        pl.BlockSpec((2, BD, 32), lambda i: (0, i, 0)),
        pl.BlockSpec((2, BD, 32), lambda i: (0, i, 0)),
        pl.BlockSpec((2, BD, 32), lambda i: (0, i, 0)),
        pl.BlockSpec((BD, D), lambda i: (i, 0)),
    ],
    out_shape=[
        jax.ShapeDtypeStruct((2, NP, 32), jnp.float32),
        jax.ShapeDtypeStruct((2, NP, 32), jnp.float32),
        jax.ShapeDtypeStruct((2, NP, 32), jnp.float32),
        jax.ShapeDtypeStruct((NP, D), jnp.float32),
    ],
)


def _ev_body(p_ref, b0_ref, ev_ref):
    ev_ref[...] = jnp.exp(p_ref[0] + p_ref[1] + b0_ref[0, 0])


_ev = pl.pallas_call(
    _ev_body,
    out_shape=jax.ShapeDtypeStruct((ROWS, W), jnp.float32),
)


def _combine_body(slf_ref, agg_ref, m_ref, seg_ref, ws_ref, out_ref):
    inv = 1.0 / (seg_ref[...] + 1e-9)
    wsum = ws_ref[0] + ws_ref[1]
    agg = jnp.concatenate([agg_ref[0], agg_ref[1]], axis=1)
    mm = jnp.concatenate([m_ref[0], m_ref[1]], axis=1)
    out_ref[...] = jnp.tanh(slf_ref[...] + agg * inv + wsum * mm)


_combine = pl.pallas_call(
    _combine_body,
    grid=(NP // BD,),
    in_specs=[
        pl.BlockSpec((BD, D), lambda i: (i, 0)),
        pl.BlockSpec((2, BD, 32), lambda i: (0, i, 0)),
        pl.BlockSpec((2, BD, 32), lambda i: (0, i, 0)),
        pl.BlockSpec((BD, 1), lambda i: (i, 0)),
        pl.BlockSpec((2, BD, 1), lambda i: (0, i, 0)),
    ],
    out_specs=pl.BlockSpec((BD, D), lambda i: (i, 0)),
    out_shape=jax.ShapeDtypeStruct((NP, D), jnp.float32),
)


@functools.partial(
    pl.kernel,
    out_type=jax.ShapeDtypeStruct((2, ROWS, W), jnp.float32),
    mesh=_SC_MESH,
    compiler_params=pltpu.CompilerParams(needs_layout_passes=False, use_tc_tiling_on_sc=False),
    scratch_types=[
        pltpu.VMEM((W,), jnp.int32),
        pltpu.VMEM((W,), jnp.int32),
        pltpu.VMEM((W,), jnp.int32),
        pltpu.VMEM((W, 32), jnp.float32),
        pltpu.VMEM((W, 32), jnp.float32),
        pltpu.VMEM((W,), jnp.float32),
        pltpu.VMEM((32,), jnp.float32),
        pltpu.VMEM((16, 16), jnp.float32),
        pltpu.VMEM_SHARED((NP, 32), jnp.float32),
        pltpu.VMEM_SHARED((DSTN, 32), jnp.float32),
        pltpu.SemaphoreType.DMA,
        pltpu.SemaphoreType.DMA,
    ],
)
def _phase1(src_hbm, dst_hbm, asrc_hbm, adst_hbm, w0_hbm, part_hbm,
            src_v, dst_v, dloc_v, bufs, bufd, sbuf, w0v, qbuf,
            asrc_sp, adst_sp, sem, sem2):
    cid = lax.axis_index("c")
    sid = lax.axis_index("s")
    pltpu.sync_copy(asrc_hbm.at[cid, pl.ds(sid * _STG, _STG)],
                    asrc_sp.at[pl.ds(sid * _STG, _STG)])
    _dstg = DSTN // 16
    pltpu.sync_copy(adst_hbm.at[cid, pl.ds(DSTLO + sid * _dstg, _dstg)],
                    adst_sp.at[pl.ds(sid * _dstg, _dstg)])
    pltpu.sync_copy(w0_hbm.at[pl.ds(cid * 32, 32)], w0v)
    plsc.subcore_barrier()
    w0a = w0v[pl.ds(0, 16)]
    w0b = w0v[pl.ds(16, 16)]
    lane = lax.broadcasted_iota(jnp.int32, (16,), 0)
    rows_per_tile = ROWS // 16
    row0 = sid * rows_per_tile

    def row_body(r, carry):
        base = row0 + r
        pltpu.sync_copy(src_hbm.at[base], src_v)
        pltpu.sync_copy(dst_hbm.at[base], dst_v)
        for g in range(W // 16):
            dloc_v[pl.ds(g * 16, 16)] = dst_v[pl.ds(g * 16, 16)] - DSTLO
        cps = pltpu.async_copy(asrc_sp.at[src_v], bufs, sem)
        cpd = pltpu.async_copy(adst_sp.at[dloc_v], bufd, sem2)
        cps.wait()
        cpd.wait()

        def grp_body(g, c):
            for j in range(16):
                e = g * 16 + j
                s0 = bufs[e, pl.ds(0, 16)] + bufd[e, pl.ds(0, 16)]
                s1 = bufs[e, pl.ds(16, 16)] + bufd[e, pl.ds(16, 16)]
                u0 = jnp.exp(s0 + s0)
                u1 = jnp.exp(s1 + s1)
                t0 = 1.0 - 2.0 / (u0 + 1.0)
                t1 = 1.0 - 2.0 / (u1 + 1.0)
                qbuf[j, pl.ds(0, 16)] = t0 * w0a + t1 * w0b
            acc = plsc.load_gather(qbuf, [lane, lane * 0])
            for d0 in range(1, 16):
                acc = acc + plsc.load_gather(qbuf, [lane, lane * 0 + d0])
            sbuf[pl.ds(g * 16, 16)] = acc
            return c

        lax.fori_loop(0, W // 16, grp_body, 0)
        pltpu.sync_copy(sbuf, part_hbm.at[cid, base])
        return carry

    lax.fori_loop(0, rows_per_tile, row_body, 0)


@functools.partial(
    pl.kernel,
    out_type=(jax.ShapeDtypeStruct((2, NP, 32), jnp.float32),
              jax.ShapeDtypeStruct((NP,), jnp.float32)),
    mesh=_SC_MESH,
    compiler_params=pltpu.CompilerParams(needs_layout_passes=False, use_tc_tiling_on_sc=False),
    scratch_types=[
        pltpu.VMEM((W,), jnp.int32),
        pltpu.VMEM((W,), jnp.int32),
        pltpu.VMEM((W,), jnp.int32),
        pltpu.VMEM((W,), jnp.float32),
        pltpu.VMEM((W, 32), jnp.float32),
        pltpu.VMEM_SHARED((NP, 32), jnp.float32),
        pltpu.VMEM_SHARED((NP,), jnp.float32),
        pltpu.VMEM_SHARED((DSTN, 32), jnp.float32),
        pltpu.SemaphoreType.DMA,
    ],
)
def _phase2(src_hbm, dst_hbm, ev_hbm, m_hbm, zagg_hbm, zseg_hbm,
            agg_out, seg_out, src_v, dst_v, dloc_v, ev_v, mbuf,
            agg_sp, seg_sp, m_sp, sem):
    cid = lax.axis_index("c")
    sid = lax.axis_index("s")
    pltpu.sync_copy(zagg_hbm.at[pl.ds(sid * _STG, _STG)],
                    agg_sp.at[pl.ds(sid * _STG, _STG)])

    @pl.when(cid == 0)
    def _():
        pltpu.sync_copy(zseg_hbm.at[pl.ds(sid * _STG, _STG)],
                        seg_sp.at[pl.ds(sid * _STG, _STG)])

    _dstg = DSTN // 16
    pltpu.sync_copy(m_hbm.at[cid, pl.ds(DSTLO + sid * _dstg, _dstg)],
                    m_sp.at[pl.ds(sid * _dstg, _dstg)])
    plsc.subcore_barrier()
    rows_per_tile = ROWS // 16
    row0 = sid * rows_per_tile

    def row_body(r, carry):
        base = row0 + r
        pltpu.sync_copy(src_hbm.at[base], src_v)
        pltpu.sync_copy(dst_hbm.at[base], dst_v)
        pltpu.sync_copy(ev_hbm.at[base], ev_v)
        for g in range(W // 16):
            dloc_v[pl.ds(g * 16, 16)] = dst_v[pl.ds(g * 16, 16)] - DSTLO
        pltpu.async_copy(m_sp.at[dloc_v], mbuf, sem).wait()

        def grp_body(g, c):
            evec = ev_v[pl.ds(g * 16, 16)]
            for j in range(16):
                e = g * 16 + j
                evs = evec[j]
                mbuf[e, pl.ds(0, 16)] = mbuf[e, pl.ds(0, 16)] * evs
                mbuf[e, pl.ds(16, 16)] = mbuf[e, pl.ds(16, 16)] * evs
            return c

        lax.fori_loop(0, W // 16, grp_body, 0)
        pltpu.sync_copy(mbuf, agg_sp.at[src_v], add=True)

        @pl.when(cid == 0)
        def _():
            pltpu.sync_copy(ev_v, seg_sp.at[src_v], add=True)

        return carry

    lax.fori_loop(0, rows_per_tile, row_body, 0)
    plsc.subcore_barrier()
    pltpu.sync_copy(agg_sp.at[pl.ds(sid * _STG, _STG)],
                    agg_out.at[cid, pl.ds(sid * _STG, _STG)])

    @pl.when(cid == 0)
    def _():
        pltpu.sync_copy(seg_sp.at[pl.ds(sid * _STG, _STG)],
                        seg_out.at[pl.ds(sid * _STG, _STG)])


@functools.partial(
    pl.kernel,
    out_type=jax.ShapeDtypeStruct((2, NP), jnp.float32),
    mesh=_SC_MESH,
    compiler_params=pltpu.CompilerParams(needs_layout_passes=False, use_tc_tiling_on_sc=False),
    scratch_types=[
        pltpu.VMEM((W,), jnp.int32),
        pltpu.VMEM((W,), jnp.float32),
        pltpu.VMEM_SHARED((NP,), jnp.float32),
    ],
)
def _wsum(src_hbm, w_hbm, zseg_hbm, out_hbm, src_v, w_v, seg_sp):
    cid = lax.axis_index("c")
    sid = lax.axis_index("s")
    pltpu.sync_copy(zseg_hbm.at[pl.ds(sid * _STG, _STG)],
                    seg_sp.at[pl.ds(sid * _STG, _STG)])
    plsc.subcore_barrier()
    wid = cid * 16 + sid
    rows_per_tile = EROWS // 32
    row0 = wid * rows_per_tile

    def row_body(r, carry):
        base = row0 + r
        pltpu.sync_copy(src_hbm.at[base], src_v)
        pltpu.sync_copy(w_hbm.at[base], w_v)
        pltpu.sync_copy(w_v, seg_sp.at[src_v], add=True)
        return carry

    lax.fori_loop(0, rows_per_tile, row_body, 0)
    plsc.subcore_barrier()
    pltpu.sync_copy(seg_sp.at[pl.ds(sid * _STG, _STG)],
                    out_hbm.at[cid, pl.ds(sid * _STG, _STG)])


def _pad_idx(n):
    return PADBASE + (jnp.arange(n, dtype=jnp.int32) % 64)


def _view(emb, er_src, er_dst, ee_src, ee_w, Wa, ba, w0, b0, Ws, bs, Wn, bn):
    x = jnp.pad(emb, ((0, NP - N), (0, 0)))
    src2d = jnp.concatenate([er_src.astype(jnp.int32),
                             _pad_idx(EP - E_ER)]).reshape(ROWS, W)
    dst2d = jnp.concatenate([er_dst.astype(jnp.int32),
                             _pad_idx(EP - E_ER)]).reshape(ROWS, W)
    esrc2d = jnp.concatenate([ee_src.astype(jnp.int32),
                              _pad_idx(EEP - E_EE)]).reshape(EROWS, W)
    ew2d = jnp.concatenate([ee_w, jnp.zeros(EEP - E_EE, jnp.float32)]
                           ).reshape(EROWS, W)
    zagg = jnp.zeros((NP, 32), jnp.float32)
    zseg = jnp.zeros((NP,), jnp.float32)
    ws3 = _wsum(esrc2d, ew2d, zseg).reshape(2, NP, 1)
    for l in range(L):
        W4 = jnp.concatenate([Wa[l][D:], Wa[l][:D], Wn[l], Ws[l]], axis=1)
        b4 = jnp.concatenate([ba[l], jnp.zeros(D, jnp.float32), bn[l], bs[l]]
                             ).reshape(1, 4 * D)
        asrc, adst, m01, slf = _dense(x, W4, b4)
        part = _phase1(src2d, dst2d, asrc, adst, w0[l][:, 0])
        ev2d = _ev(part, b0[l].reshape(1, 1))
        agg, seg = _phase2(src2d, dst2d, ev2d, m01, zagg, zseg)
        x = _combine(slf, agg, m01, seg.reshape(NP, 1), ws3)
    return x


def kernel(er_src_H, er_dst_H, ee_src_H, ee_weight_H,
           er_src_T, er_dst_T, ee_src_T, ee_weight_T,
           embH, embT,
           WH_attn, bH_attn, wH_0, bH_0, WH_self, bH_self, WH_neigh, bH_neigh,
           WT_attn, bT_attn, wT_0, bT_0, WT_self, bT_self, WT_neigh, bT_neigh):
    xH = _view(embH, er_src_H, er_dst_H, ee_src_H, ee_weight_H,
               WH_attn, bH_attn, wH_0, bH_0, WH_self, bH_self,
               WH_neigh, bH_neigh)
    xT = _view(embT, er_src_T, er_dst_T, ee_src_T, ee_weight_T,
               WT_attn, bT_attn, wT_0, bT_0, WT_self, bT_self,
               WT_neigh, bT_neigh)
    return (xH[:NE], xH[NE:N], xT[:NE], xT[NE:N])


# speedup vs baseline: 7.2836x; 1.2881x over previous
import functools

import jax
import jax.numpy as jnp
from jax import lax
from jax.experimental import pallas as pl
from jax.experimental.pallas import tpu as pltpu
from jax.experimental.pallas import tpu_sc as plsc

NE = 50000
NR = 1000
N = NE + NR
D = 64
L = 2
NP = 51200
E_ER = 800000
E_EE = 200000
W = 128
ROWS = 6400
EROWS = 1600
EP = ROWS * W
EEP = EROWS * W
DSTLO = NE
DSTN = 1088
PADBASE = 51008
BD = 1024

_SC_MESH = plsc.VectorSubcoreMesh(core_axis_name="c", subcore_axis_name="s")
_STG = NP // 16
B1 = 2
B2 = 4
BW = 10


def _dense_body(x_ref, w_ref, b_ref, asrc_ref, adst_ref, m_ref, slf_ref):
    y = jnp.dot(x_ref[...], w_ref[...], preferred_element_type=jnp.float32)
    y = y + b_ref[...]
    asrc_ref[...] = jnp.stack([y[:, 0:32], y[:, 32:64]])
    adst_ref[...] = jnp.stack([y[:, 64:96], y[:, 96:128]])
    m_ref[...] = jnp.stack([y[:, 128:160], y[:, 160:192]])
    slf_ref[...] = y[:, 192:256]


_dense = pl.pallas_call(
    _dense_body,
    grid=(NP // BD,),
    in_specs=[
        pl.BlockSpec((BD, D), lambda i: (i, 0)),
        pl.BlockSpec((D, 4 * D), lambda i: (0, 0)),
        pl.BlockSpec((1, 4 * D), lambda i: (0, 0)),
    ],
    out_specs=[
        pl.BlockSpec((2, BD, 32), lambda i: (0, i, 0)),
        pl.BlockSpec((2, BD, 32), lambda i: (0, i, 0)),
        pl.BlockSpec((2, BD, 32), lambda i: (0, i, 0)),
        pl.BlockSpec((BD, D), lambda i: (i, 0)),
    ],
    out_shape=[
        jax.ShapeDtypeStruct((2, NP, 32), jnp.float32),
        jax.ShapeDtypeStruct((2, NP, 32), jnp.float32),
        jax.ShapeDtypeStruct((2, NP, 32), jnp.float32),
        jax.ShapeDtypeStruct((NP, D), jnp.float32),
    ],
)


def _ev_body(p_ref, b0_ref, ev_ref):
    ev_ref[...] = jnp.exp(p_ref[0] + p_ref[1] + b0_ref[0, 0])


_ev = pl.pallas_call(
    _ev_body,
    out_shape=jax.ShapeDtypeStruct((ROWS, W), jnp.float32),
)


def _combine_body(slf_ref, agg_ref, m_ref, seg_ref, ws_ref, out_ref):
    inv = 1.0 / (seg_ref[...] + 1e-9)
    wsum = ws_ref[0] + ws_ref[1]
    agg = jnp.concatenate([agg_ref[0], agg_ref[1]], axis=1)
    mm = jnp.concatenate([m_ref[0], m_ref[1]], axis=1)
    out_ref[...] = jnp.tanh(slf_ref[...] + agg * inv + wsum * mm)


_combine = pl.pallas_call(
    _combine_body,
    grid=(NP // BD,),
    in_specs=[
        pl.BlockSpec((BD, D), lambda i: (i, 0)),
        pl.BlockSpec((2, BD, 32), lambda i: (0, i, 0)),
        pl.BlockSpec((2, BD, 32), lambda i: (0, i, 0)),
        pl.BlockSpec((BD, 1), lambda i: (i, 0)),
        pl.BlockSpec((2, BD, 1), lambda i: (0, i, 0)),
    ],
    out_specs=pl.BlockSpec((BD, D), lambda i: (i, 0)),
    out_shape=jax.ShapeDtypeStruct((NP, D), jnp.float32),
)


@functools.partial(
    pl.kernel,
    out_type=jax.ShapeDtypeStruct((2, ROWS, W), jnp.float32),
    mesh=_SC_MESH,
    compiler_params=pltpu.CompilerParams(needs_layout_passes=False, use_tc_tiling_on_sc=False),
    scratch_types=[
        pltpu.VMEM((B1, W), jnp.int32),
        pltpu.VMEM((B1, W), jnp.int32),
        pltpu.VMEM((B1, W), jnp.int32),
        pltpu.VMEM((B1 * W, 32), jnp.float32),
        pltpu.VMEM((B1 * W, 32), jnp.float32),
        pltpu.VMEM((B1, W), jnp.float32),
        pltpu.VMEM((32,), jnp.float32),
        pltpu.VMEM((16, 16), jnp.float32),
        pltpu.VMEM_SHARED((NP, 32), jnp.float32),
        pltpu.VMEM_SHARED((DSTN, 32), jnp.float32),
        pltpu.SemaphoreType.DMA,
    ],
)
def _phase1(src_hbm, dst_hbm, asrc_hbm, adst_hbm, w0_hbm, part_hbm,
            srcm, dstm, dlocm, bufs, bufd, sbuf, w0v, qbuf,
            asrc_sp, adst_sp, sem):
    cid = lax.axis_index("c")
    sid = lax.axis_index("s")
    pltpu.sync_copy(asrc_hbm.at[cid, pl.ds(sid * _STG, _STG)],
                    asrc_sp.at[pl.ds(sid * _STG, _STG)])
    _dstg = DSTN // 16
    pltpu.sync_copy(adst_hbm.at[cid, pl.ds(DSTLO + sid * _dstg, _dstg)],
                    adst_sp.at[pl.ds(sid * _dstg, _dstg)])
    pltpu.sync_copy(w0_hbm.at[pl.ds(cid * 32, 32)], w0v)
    plsc.subcore_barrier()
    w0a = w0v[pl.ds(0, 16)]
    w0b = w0v[pl.ds(16, 16)]
    lane = lax.broadcasted_iota(jnp.int32, (16,), 0)
    rows_per_tile = ROWS // 16
    row0 = sid * rows_per_tile

    def it_body(t, carry):
        base = row0 + t * B1
        pltpu.sync_copy(src_hbm.at[pl.ds(base, B1)], srcm)
        pltpu.sync_copy(dst_hbm.at[pl.ds(base, B1)], dstm)
        for r2 in range(B1):
            for g in range(W // 16):
                dlocm[r2, pl.ds(g * 16, 16)] = (
                    dstm[r2, pl.ds(g * 16, 16)] - DSTLO)
        hs = []
        for k in range(B1):
            hs.append(pltpu.async_copy(
                asrc_sp.at[srcm.at[k]], bufs.at[pl.ds(k * W, W)], sem))
            hs.append(pltpu.async_copy(
                adst_sp.at[dlocm.at[k]], bufd.at[pl.ds(k * W, W)], sem))
        for h in hs:
            h.wait()

        def rcomp(r2, c):
            def gcomp(g, c2):
                e0 = r2 * W + g * 16
                for j in range(16):
                    e = e0 + j
                    s0 = bufs[e, pl.ds(0, 16)] + bufd[e, pl.ds(0, 16)]
                    s1 = bufs[e, pl.ds(16, 16)] + bufd[e, pl.ds(16, 16)]
                    u0 = jnp.exp(s0 + s0)
                    u1 = jnp.exp(s1 + s1)
                    t0 = 1.0 - 2.0 / (u0 + 1.0)
                    t1 = 1.0 - 2.0 / (u1 + 1.0)
                    qbuf[j, pl.ds(0, 16)] = t0 * w0a + t1 * w0b
                acc = plsc.load_gather(qbuf, [lane, lane * 0])
                for d0 in range(1, 16):
                    acc = acc + plsc.load_gather(qbuf, [lane, lane * 0 + d0])
                sbuf[r2, pl.ds(g * 16, 16)] = acc
                return c2

            lax.fori_loop(0, W // 16, gcomp, 0)
            return c

        lax.fori_loop(0, B1, rcomp, 0)
        pltpu.sync_copy(sbuf, part_hbm.at[cid, pl.ds(base, B1)])
        return carry

    lax.fori_loop(0, rows_per_tile // B1, it_body, 0)


@functools.partial(
    pl.kernel,
    out_type=(jax.ShapeDtypeStruct((2, NP, 32), jnp.float32),
              jax.ShapeDtypeStruct((NP,), jnp.float32)),
    mesh=_SC_MESH,
    compiler_params=pltpu.CompilerParams(needs_layout_passes=False, use_tc_tiling_on_sc=False),
    scratch_types=[
        pltpu.VMEM((B2, W), jnp.int32),
        pltpu.VMEM((B2, W), jnp.int32),
        pltpu.VMEM((B2, W), jnp.int32),
        pltpu.VMEM((B2, W), jnp.float32),
        pltpu.VMEM((B2 * W, 32), jnp.float32),
        pltpu.VMEM_SHARED((NP, 32), jnp.float32),
        pltpu.VMEM_SHARED((NP,), jnp.float32),
        pltpu.VMEM_SHARED((DSTN, 32), jnp.float32),
        pltpu.SemaphoreType.DMA,
        pltpu.SemaphoreType.DMA,
    ],
)
def _phase2(src_hbm, dst_hbm, ev_hbm, m_hbm, zagg_hbm, zseg_hbm,
            agg_out, seg_out, srcm, dstm, dlocm, evm, mbuf,
            agg_sp, seg_sp, m_sp, sem, sem2):
    cid = lax.axis_index("c")
    sid = lax.axis_index("s")
    pltpu.sync_copy(zagg_hbm.at[pl.ds(sid * _STG, _STG)],
                    agg_sp.at[pl.ds(sid * _STG, _STG)])

    @pl.when(cid == 0)
    def _():
        pltpu.sync_copy(zseg_hbm.at[pl.ds(sid * _STG, _STG)],
                        seg_sp.at[pl.ds(sid * _STG, _STG)])

    _dstg = DSTN // 16
    pltpu.sync_copy(m_hbm.at[cid, pl.ds(DSTLO + sid * _dstg, _dstg)],
                    m_sp.at[pl.ds(sid * _dstg, _dstg)])
    plsc.subcore_barrier()
    rows_per_tile = ROWS // 16
    row0 = sid * rows_per_tile

    def it_body(t, carry):
        base = row0 + t * B2
        pltpu.sync_copy(src_hbm.at[pl.ds(base, B2)], srcm)
        pltpu.sync_copy(dst_hbm.at[pl.ds(base, B2)], dstm)
        pltpu.sync_copy(ev_hbm.at[pl.ds(base, B2)], evm)
        for r2 in range(B2):
            for g in range(W // 16):
                dlocm[r2, pl.ds(g * 16, 16)] = (
                    dstm[r2, pl.ds(g * 16, 16)] - DSTLO)
        hs = []
        for k in range(B2):
            hs.append(pltpu.async_copy(
                m_sp.at[dlocm.at[k]], mbuf.at[pl.ds(k * W, W)], sem))
        for h in hs:
            h.wait()

        def rcomp(r2, c):
            def gcomp(g, c2):
                evec = evm[r2, pl.ds(g * 16, 16)]
                e0 = r2 * W + g * 16
                for j in range(16):
                    e = e0 + j
                    evs = evec[j]
                    mbuf[e, pl.ds(0, 16)] = mbuf[e, pl.ds(0, 16)] * evs
                    mbuf[e, pl.ds(16, 16)] = mbuf[e, pl.ds(16, 16)] * evs
                return c2

            lax.fori_loop(0, W // 16, gcomp, 0)
            return c

        lax.fori_loop(0, B2, rcomp, 0)
        hs2 = []
        for k in range(B2):
            hs2.append(pltpu.async_copy(
                mbuf.at[pl.ds(k * W, W)], agg_sp.at[srcm.at[k]], sem2,
                add=True))

        @pl.when(cid == 0)
        def _():
            evhs = []
            for k in range(B2):
                evhs.append(pltpu.async_copy(
                    evm.at[k], seg_sp.at[srcm.at[k]], sem2, add=True))
            for h in evhs:
                h.wait()

        for h in hs2:
            h.wait()
        return carry

    lax.fori_loop(0, rows_per_tile // B2, it_body, 0)
    plsc.subcore_barrier()
    pltpu.sync_copy(agg_sp.at[pl.ds(sid * _STG, _STG)],
                    agg_out.at[cid, pl.ds(sid * _STG, _STG)])

    @pl.when(cid == 0)
    def _():
        pltpu.sync_copy(seg_sp.at[pl.ds(sid * _STG, _STG)],
                        seg_out.at[pl.ds(sid * _STG, _STG)])


@functools.partial(
    pl.kernel,
    out_type=jax.ShapeDtypeStruct((2, NP), jnp.float32),
    mesh=_SC_MESH,
    compiler_params=pltpu.CompilerParams(needs_layout_passes=False, use_tc_tiling_on_sc=False),
    scratch_types=[
        pltpu.VMEM((BW, W), jnp.int32),
        pltpu.VMEM((BW, W), jnp.float32),
        pltpu.VMEM_SHARED((NP,), jnp.float32),
        pltpu.SemaphoreType.DMA,
    ],
)
def _wsum(src_hbm, w_hbm, zseg_hbm, out_hbm, srcm, wm, seg_sp, sem):
    cid = lax.axis_index("c")
    sid = lax.axis_index("s")
    pltpu.sync_copy(zseg_hbm.at[pl.ds(sid * _STG, _STG)],
                    seg_sp.at[pl.ds(sid * _STG, _STG)])
    plsc.subcore_barrier()
    wid = cid * 16 + sid
    rows_per_tile = EROWS // 32
    row0 = wid * rows_per_tile

    def it_body(t, carry):
        base = row0 + t * BW
        pltpu.sync_copy(src_hbm.at[pl.ds(base, BW)], srcm)
        pltpu.sync_copy(w_hbm.at[pl.ds(base, BW)], wm)
        hs = []
        for k in range(BW):
            hs.append(pltpu.async_copy(
                wm.at[k], seg_sp.at[srcm.at[k]], sem, add=True))
        for h in hs:
            h.wait()
        return carry

    lax.fori_loop(0, rows_per_tile // BW, it_body, 0)
    plsc.subcore_barrier()
    pltpu.sync_copy(seg_sp.at[pl.ds(sid * _STG, _STG)],
                    out_hbm.at[cid, pl.ds(sid * _STG, _STG)])


def _pad_idx(n):
    return PADBASE + (jnp.arange(n, dtype=jnp.int32) % 64)


def _view(emb, er_src, er_dst, ee_src, ee_w, Wa, ba, w0, b0, Ws, bs, Wn, bn):
    x = jnp.pad(emb, ((0, NP - N), (0, 0)))
    src2d = jnp.concatenate([er_src.astype(jnp.int32),
                             _pad_idx(EP - E_ER)]).reshape(ROWS, W)
    dst2d = jnp.concatenate([er_dst.astype(jnp.int32),
                             _pad_idx(EP - E_ER)]).reshape(ROWS, W)
    esrc2d = jnp.concatenate([ee_src.astype(jnp.int32),
                              _pad_idx(EEP - E_EE)]).reshape(EROWS, W)
    ew2d = jnp.concatenate([ee_w, jnp.zeros(EEP - E_EE, jnp.float32)]
                           ).reshape(EROWS, W)
    zagg = jnp.zeros((NP, 32), jnp.float32)
    zseg = jnp.zeros((NP,), jnp.float32)
    ws3 = _wsum(esrc2d, ew2d, zseg).reshape(2, NP, 1)
    for l in range(L):
        W4 = jnp.concatenate([Wa[l][D:], Wa[l][:D], Wn[l], Ws[l]], axis=1)
        b4 = jnp.concatenate([ba[l], jnp.zeros(D, jnp.float32), bn[l], bs[l]]
                             ).reshape(1, 4 * D)
        asrc, adst, m01, slf = _dense(x, W4, b4)
        part = _phase1(src2d, dst2d, asrc, adst, w0[l][:, 0])
        ev2d = _ev(part, b0[l].reshape(1, 1))
        agg, seg = _phase2(src2d, dst2d, ev2d, m01, zagg, zseg)
        x = _combine(slf, agg, m01, seg.reshape(NP, 1), ws3)
    return x


def kernel(er_src_H, er_dst_H, ee_src_H, ee_weight_H,
           er_src_T, er_dst_T, ee_src_T, ee_weight_T,
           embH, embT,
           WH_attn, bH_attn, wH_0, bH_0, WH_self, bH_self, WH_neigh, bH_neigh,
           WT_attn, bT_attn, wT_0, bT_0, WT_self, bT_self, WT_neigh, bT_neigh):
    xH = _view(embH, er_src_H, er_dst_H, ee_src_H, ee_weight_H,
               WH_attn, bH_attn, wH_0, bH_0, WH_self, bH_self,
               WH_neigh, bH_neigh)
    xT = _view(embT, er_src_T, er_dst_T, ee_src_T, ee_weight_T,
               WT_attn, bT_attn, wT_0, bT_0, WT_self, bT_self,
               WT_neigh, bT_neigh)
    return (xH[:NE], xH[NE:N], xT[:NE], xT[NE:N])


# speedup vs baseline: 9.8636x; 1.3542x over previous
import functools

import jax
import jax.numpy as jnp
from jax import lax
from jax.experimental import pallas as pl
from jax.experimental.pallas import tpu as pltpu
from jax.experimental.pallas import tpu_sc as plsc

NE = 50000
NR = 1000
N = NE + NR
D = 64
L = 2
NP = 51200
E_ER = 800000
E_EE = 200000
W = 128
ROWS = 6400
EROWS = 1600
EP = ROWS * W
EEP = EROWS * W
DSTLO = NE
DSTN = 1088
PADBASE = 51008
BD = 1024

_SC_MESH = plsc.VectorSubcoreMesh(core_axis_name="c", subcore_axis_name="s")
_STG = NP // 16
B1 = 2
B2 = 4
BW = 10


def _dense_body(x_ref, w_ref, b_ref, asrc_ref, adst_ref, m_ref, slf_ref):
    y = jnp.dot(x_ref[...], w_ref[...], preferred_element_type=jnp.float32)
    y = y + b_ref[...]
    asrc_ref[...] = jnp.stack([y[:, 0:32], y[:, 32:64]])
    adst_ref[...] = jnp.stack([y[:, 64:96], y[:, 96:128]])
    m_ref[...] = jnp.stack([y[:, 128:160], y[:, 160:192]])
    slf_ref[...] = y[:, 192:256]


_dense = pl.pallas_call(
    _dense_body,
    grid=(NP // BD,),
    in_specs=[
        pl.BlockSpec((BD, D), lambda i: (i, 0)),
        pl.BlockSpec((D, 4 * D), lambda i: (0, 0)),
        pl.BlockSpec((1, 4 * D), lambda i: (0, 0)),
    ],
    out_specs=[
        pl.BlockSpec((2, BD, 32), lambda i: (0, i, 0)),
        pl.BlockSpec((2, BD, 32), lambda i: (0, i, 0)),
        pl.BlockSpec((2, BD, 32), lambda i: (0, i, 0)),
        pl.BlockSpec((BD, D), lambda i: (i, 0)),
    ],
    out_shape=[
        jax.ShapeDtypeStruct((2, NP, 32), jnp.float32),
        jax.ShapeDtypeStruct((2, NP, 32), jnp.float32),
        jax.ShapeDtypeStruct((2, NP, 32), jnp.float32),
        jax.ShapeDtypeStruct((NP, D), jnp.float32),
    ],
)


BDE = 8192


def _ev_body(s_ref, w0_ref, b0_ref, ev_ref):
    t0 = jnp.tanh(s_ref[0])
    t1 = jnp.tanh(s_ref[1])
    logit = (jnp.dot(t0, w0_ref[0:32], preferred_element_type=jnp.float32)
             + jnp.dot(t1, w0_ref[32:64], preferred_element_type=jnp.float32))
    ev_ref[...] = jnp.exp(logit + b0_ref[0, 0]).reshape(BDE // W, W)


_ev = pl.pallas_call(
    _ev_body,
    grid=(EP // BDE,),
    in_specs=[
        pl.BlockSpec((2, BDE, 32), lambda i: (0, i, 0)),
        pl.BlockSpec((D, 1), lambda i: (0, 0)),
        pl.BlockSpec((1, 1), lambda i: (0, 0)),
    ],
    out_specs=pl.BlockSpec((BDE // W, W), lambda i: (i, 0)),
    out_shape=jax.ShapeDtypeStruct((ROWS, W), jnp.float32),
)


def _combine_body(slf_ref, agg_ref, m_ref, seg_ref, ws_ref, out_ref):
    inv = 1.0 / (seg_ref[...] + 1e-9)
    wsum = ws_ref[0] + ws_ref[1]
    agg = jnp.concatenate([agg_ref[0], agg_ref[1]], axis=1)
    mm = jnp.concatenate([m_ref[0], m_ref[1]], axis=1)
    out_ref[...] = jnp.tanh(slf_ref[...] + agg * inv + wsum * mm)


_combine = pl.pallas_call(
    _combine_body,
    grid=(NP // BD,),
    in_specs=[
        pl.BlockSpec((BD, D), lambda i: (i, 0)),
        pl.BlockSpec((2, BD, 32), lambda i: (0, i, 0)),
        pl.BlockSpec((2, BD, 32), lambda i: (0, i, 0)),
        pl.BlockSpec((BD, 1), lambda i: (i, 0)),
        pl.BlockSpec((2, BD, 1), lambda i: (0, i, 0)),
    ],
    out_specs=pl.BlockSpec((BD, D), lambda i: (i, 0)),
    out_shape=jax.ShapeDtypeStruct((NP, D), jnp.float32),
)


@functools.partial(
    pl.kernel,
    out_type=jax.ShapeDtypeStruct((2, EP, 32), jnp.float32),
    mesh=_SC_MESH,
    compiler_params=pltpu.CompilerParams(needs_layout_passes=False, use_tc_tiling_on_sc=False),
    scratch_types=[
        pltpu.VMEM((B1, W), jnp.int32),
        pltpu.VMEM((B1, W), jnp.int32),
        pltpu.VMEM((B1, W), jnp.int32),
        pltpu.VMEM((B1 * W, 32), jnp.float32),
        pltpu.VMEM((B1 * W, 32), jnp.float32),
        pltpu.VMEM_SHARED((NP, 32), jnp.float32),
        pltpu.VMEM_SHARED((DSTN, 32), jnp.float32),
        pltpu.SemaphoreType.DMA,
    ],
)
def _phase1(src_hbm, dst_hbm, asrc_hbm, adst_hbm, s_hbm,
            srcm, dstm, dlocm, bufs, bufd,
            asrc_sp, adst_sp, sem):
    cid = lax.axis_index("c")
    sid = lax.axis_index("s")
    pltpu.sync_copy(asrc_hbm.at[cid, pl.ds(sid * _STG, _STG)],
                    asrc_sp.at[pl.ds(sid * _STG, _STG)])
    _dstg = DSTN // 16
    pltpu.sync_copy(adst_hbm.at[cid, pl.ds(DSTLO + sid * _dstg, _dstg)],
                    adst_sp.at[pl.ds(sid * _dstg, _dstg)])
    plsc.subcore_barrier()
    rows_per_tile = ROWS // 16
    row0 = sid * rows_per_tile

    def it_body(t, carry):
        base = row0 + t * B1
        pltpu.sync_copy(src_hbm.at[pl.ds(base, B1)], srcm)
        pltpu.sync_copy(dst_hbm.at[pl.ds(base, B1)], dstm)
        for r2 in range(B1):
            for g in range(W // 16):
                dlocm[r2, pl.ds(g * 16, 16)] = (
                    dstm[r2, pl.ds(g * 16, 16)] - DSTLO)
        hs = []
        for k in range(B1):
            hs.append(pltpu.async_copy(
                asrc_sp.at[srcm.at[k]], bufs.at[pl.ds(k * W, W)], sem))
            hs.append(pltpu.async_copy(
                adst_sp.at[dlocm.at[k]], bufd.at[pl.ds(k * W, W)], sem))
        for h in hs:
            h.wait()

        def rcomp(r2, c):
            def gcomp(g, c2):
                e0 = r2 * W + g * 16
                for j in range(16):
                    e = e0 + j
                    bufs[e, pl.ds(0, 16)] = (
                        bufs[e, pl.ds(0, 16)] + bufd[e, pl.ds(0, 16)])
                    bufs[e, pl.ds(16, 16)] = (
                        bufs[e, pl.ds(16, 16)] + bufd[e, pl.ds(16, 16)])
                return c2

            lax.fori_loop(0, W // 16, gcomp, 0)
            return c

        lax.fori_loop(0, B1, rcomp, 0)
        pltpu.sync_copy(bufs, s_hbm.at[cid, pl.ds(base * W, B1 * W)])
        return carry

    lax.fori_loop(0, rows_per_tile // B1, it_body, 0)


@functools.partial(
    pl.kernel,
    out_type=(jax.ShapeDtypeStruct((2, NP, 32), jnp.float32),
              jax.ShapeDtypeStruct((NP,), jnp.float32)),
    mesh=_SC_MESH,
    compiler_params=pltpu.CompilerParams(needs_layout_passes=False, use_tc_tiling_on_sc=False),
    scratch_types=[
        pltpu.VMEM((B2, W), jnp.int32),
        pltpu.VMEM((B2, W), jnp.int32),
        pltpu.VMEM((B2, W), jnp.int32),
        pltpu.VMEM((B2, W), jnp.float32),
        pltpu.VMEM((B2 * W, 32), jnp.float32),
        pltpu.VMEM_SHARED((NP, 32), jnp.float32),
        pltpu.VMEM_SHARED((NP,), jnp.float32),
        pltpu.VMEM_SHARED((DSTN, 32), jnp.float32),
        pltpu.SemaphoreType.DMA,
        pltpu.SemaphoreType.DMA,
    ],
)
def _phase2(src_hbm, dst_hbm, ev_hbm, m_hbm, zagg_hbm, zseg_hbm,
            agg_out, seg_out, srcm, dstm, dlocm, evm, mbuf,
            agg_sp, seg_sp, m_sp, sem, sem2):
    cid = lax.axis_index("c")
    sid = lax.axis_index("s")
    pltpu.sync_copy(zagg_hbm.at[pl.ds(sid * _STG, _STG)],
                    agg_sp.at[pl.ds(sid * _STG, _STG)])

    @pl.when(cid == 0)
    def _():
        pltpu.sync_copy(zseg_hbm.at[pl.ds(sid * _STG, _STG)],
                        seg_sp.at[pl.ds(sid * _STG, _STG)])

    _dstg = DSTN // 16
    pltpu.sync_copy(m_hbm.at[cid, pl.ds(DSTLO + sid * _dstg, _dstg)],
                    m_sp.at[pl.ds(sid * _dstg, _dstg)])
    plsc.subcore_barrier()
    rows_per_tile = ROWS // 16
    row0 = sid * rows_per_tile

    def it_body(t, carry):
        base = row0 + t * B2
        pltpu.sync_copy(src_hbm.at[pl.ds(base, B2)], srcm)
        pltpu.sync_copy(dst_hbm.at[pl.ds(base, B2)], dstm)
        pltpu.sync_copy(ev_hbm.at[pl.ds(base, B2)], evm)
        for r2 in range(B2):
            for g in range(W // 16):
                dlocm[r2, pl.ds(g * 16, 16)] = (
                    dstm[r2, pl.ds(g * 16, 16)] - DSTLO)
        hs = []
        for k in range(B2):
            hs.append(pltpu.async_copy(
                m_sp.at[dlocm.at[k]], mbuf.at[pl.ds(k * W, W)], sem))
        for h in hs:
            h.wait()

        def rcomp(r2, c):
            def gcomp(g, c2):
                evec = evm[r2, pl.ds(g * 16, 16)]
                e0 = r2 * W + g * 16
                for j in range(16):
                    e = e0 + j
                    evs = evec[j]
                    mbuf[e, pl.ds(0, 16)] = mbuf[e, pl.ds(0, 16)] * evs
                    mbuf[e, pl.ds(16, 16)] = mbuf[e, pl.ds(16, 16)] * evs
                return c2

            lax.fori_loop(0, W // 16, gcomp, 0)
            return c

        lax.fori_loop(0, B2, rcomp, 0)
        hs2 = []
        for k in range(B2):
            hs2.append(pltpu.async_copy(
                mbuf.at[pl.ds(k * W, W)], agg_sp.at[srcm.at[k]], sem2,
                add=True))

        @pl.when(cid == 0)
        def _():
            evhs = []
            for k in range(B2):
                evhs.append(pltpu.async_copy(
                    evm.at[k], seg_sp.at[srcm.at[k]], sem2, add=True))
            for h in evhs:
                h.wait()

        for h in hs2:
            h.wait()
        return carry

    lax.fori_loop(0, rows_per_tile // B2, it_body, 0)
    plsc.subcore_barrier()
    pltpu.sync_copy(agg_sp.at[pl.ds(sid * _STG, _STG)],
                    agg_out.at[cid, pl.ds(sid * _STG, _STG)])

    @pl.when(cid == 0)
    def _():
        pltpu.sync_copy(seg_sp.at[pl.ds(sid * _STG, _STG)],
                        seg_out.at[pl.ds(sid * _STG, _STG)])


@functools.partial(
    pl.kernel,
    out_type=jax.ShapeDtypeStruct((2, NP), jnp.float32),
    mesh=_SC_MESH,
    compiler_params=pltpu.CompilerParams(needs_layout_passes=False, use_tc_tiling_on_sc=False),
    scratch_types=[
        pltpu.VMEM((BW, W), jnp.int32),
        pltpu.VMEM((BW, W), jnp.float32),
        pltpu.VMEM_SHARED((NP,), jnp.float32),
        pltpu.SemaphoreType.DMA,
    ],
)
def _wsum(src_hbm, w_hbm, zseg_hbm, out_hbm, srcm, wm, seg_sp, sem):
    cid = lax.axis_index("c")
    sid = lax.axis_index("s")
    pltpu.sync_copy(zseg_hbm.at[pl.ds(sid * _STG, _STG)],
                    seg_sp.at[pl.ds(sid * _STG, _STG)])
    plsc.subcore_barrier()
    wid = cid * 16 + sid
    rows_per_tile = EROWS // 32
    row0 = wid * rows_per_tile

    def it_body(t, carry):
        base = row0 + t * BW
        pltpu.sync_copy(src_hbm.at[pl.ds(base, BW)], srcm)
        pltpu.sync_copy(w_hbm.at[pl.ds(base, BW)], wm)
        hs = []
        for k in range(BW):
            hs.append(pltpu.async_copy(
                wm.at[k], seg_sp.at[srcm.at[k]], sem, add=True))
        for h in hs:
            h.wait()
        return carry

    lax.fori_loop(0, rows_per_tile // BW, it_body, 0)
    plsc.subcore_barrier()
    pltpu.sync_copy(seg_sp.at[pl.ds(sid * _STG, _STG)],
                    out_hbm.at[cid, pl.ds(sid * _STG, _STG)])


def _pad_idx(n):
    return PADBASE + (jnp.arange(n, dtype=jnp.int32) % 64)


def _view(emb, er_src, er_dst, ee_src, ee_w, Wa, ba, w0, b0, Ws, bs, Wn, bn):
    x = jnp.pad(emb, ((0, NP - N), (0, 0)))
    src2d = jnp.concatenate([er_src.astype(jnp.int32),
                             _pad_idx(EP - E_ER)]).reshape(ROWS, W)
    dst2d = jnp.concatenate([er_dst.astype(jnp.int32),
                             _pad_idx(EP - E_ER)]).reshape(ROWS, W)
    esrc2d = jnp.concatenate([ee_src.astype(jnp.int32),
                              _pad_idx(EEP - E_EE)]).reshape(EROWS, W)
    ew2d = jnp.concatenate([ee_w, jnp.zeros(EEP - E_EE, jnp.float32)]
                           ).reshape(EROWS, W)
    zagg = jnp.zeros((NP, 32), jnp.float32)
    zseg = jnp.zeros((NP,), jnp.float32)
    ws3 = _wsum(esrc2d, ew2d, zseg).reshape(2, NP, 1)
    for l in range(L):
        W4 = jnp.concatenate([Wa[l][D:], Wa[l][:D], Wn[l], Ws[l]], axis=1)
        b4 = jnp.concatenate([ba[l], jnp.zeros(D, jnp.float32), bn[l], bs[l]]
                             ).reshape(1, 4 * D)
        asrc, adst, m01, slf = _dense(x, W4, b4)
        s01 = _phase1(src2d, dst2d, asrc, adst)
        ev2d = _ev(s01, w0[l].reshape(D, 1), b0[l].reshape(1, 1))
        agg, seg = _phase2(src2d, dst2d, ev2d, m01, zagg, zseg)
        x = _combine(slf, agg, m01, seg.reshape(NP, 1), ws3)
    return x


def kernel(er_src_H, er_dst_H, ee_src_H, ee_weight_H,
           er_src_T, er_dst_T, ee_src_T, ee_weight_T,
           embH, embT,
           WH_attn, bH_attn, wH_0, bH_0, WH_self, bH_self, WH_neigh, bH_neigh,
           WT_attn, bT_attn, wT_0, bT_0, WT_self, bT_self, WT_neigh, bT_neigh):
    xH = _view(embH, er_src_H, er_dst_H, ee_src_H, ee_weight_H,
               WH_attn, bH_attn, wH_0, bH_0, WH_self, bH_self,
               WH_neigh, bH_neigh)
    xT = _view(embT, er_src_T, er_dst_T, ee_src_T, ee_weight_T,
               WT_attn, bT_attn, wT_0, bT_0, WT_self, bT_self,
               WT_neigh, bT_neigh)
    return (xH[:NE], xH[NE:N], xT[:NE], xT[NE:N])


# speedup vs baseline: 12.3976x; 1.2569x over previous
import functools

import jax
import jax.numpy as jnp
from jax import lax
from jax.experimental import pallas as pl
from jax.experimental.pallas import tpu as pltpu
from jax.experimental.pallas import tpu_sc as plsc

NE = 50000
NR = 1000
N = NE + NR
D = 64
L = 2
NP = 51200
E_ER = 800000
E_EE = 200000
W = 128
ROWS = 6400
EROWS = 1600
EP = ROWS * W
EEP = EROWS * W
DSTLO = NE
DSTN = 1088
PADBASE = 51008
BD = 1024

_SC_MESH = plsc.VectorSubcoreMesh(core_axis_name="c", subcore_axis_name="s")
_STG = NP // 16
B1 = 2
IC1 = 8
B2 = 4
BW = 10


def _dense_body(x_ref, w_ref, b_ref, asrc_ref, adst_ref, m_ref, slf_ref):
    y = jnp.dot(x_ref[...], w_ref[...], preferred_element_type=jnp.float32)
    y = y + b_ref[...]
    asrc_ref[...] = jnp.stack([y[:, 0:32], y[:, 32:64]])
    adst_ref[...] = jnp.stack([y[:, 64:96], y[:, 96:128]])
    m_ref[...] = jnp.stack([y[:, 128:160], y[:, 160:192]])
    slf_ref[...] = y[:, 192:256]


_dense = pl.pallas_call(
    _dense_body,
    grid=(NP // BD,),
    in_specs=[
        pl.BlockSpec((BD, D), lambda i: (i, 0)),
        pl.BlockSpec((D, 4 * D), lambda i: (0, 0)),
        pl.BlockSpec((1, 4 * D), lambda i: (0, 0)),
    ],
    out_specs=[
        pl.BlockSpec((2, BD, 32), lambda i: (0, i, 0)),
        pl.BlockSpec((2, BD, 32), lambda i: (0, i, 0)),
        pl.BlockSpec((2, BD, 32), lambda i: (0, i, 0)),
        pl.BlockSpec((BD, D), lambda i: (i, 0)),
    ],
    out_shape=[
        jax.ShapeDtypeStruct((2, NP, 32), jnp.float32),
        jax.ShapeDtypeStruct((2, NP, 32), jnp.float32),
        jax.ShapeDtypeStruct((2, NP, 32), jnp.float32),
        jax.ShapeDtypeStruct((NP, D), jnp.float32),
    ],
)


BDE = 8192


def _ev_body(s_ref, w0a_ref, w0b_ref, b0_ref, ev_ref):
    q = (jnp.tanh(s_ref[0]) * w0a_ref[...]
         + jnp.tanh(s_ref[1]) * w0b_ref[...])
    lane = lax.broadcasted_iota(jnp.int32, (W, 4), 0)
    col = lax.broadcasted_iota(jnp.int32, (W, 4), 1)
    sel = jnp.where(lane // 32 == col, 1.0, 0.0)
    logit4 = jnp.dot(q, sel, preferred_element_type=jnp.float32)
    ev_ref[...] = jnp.exp(logit4 + b0_ref[0, 0])


_ev = pl.pallas_call(
    _ev_body,
    grid=(EP // BDE,),
    in_specs=[
        pl.BlockSpec((2, BDE // 4, W), lambda i: (0, i, 0)),
        pl.BlockSpec((1, W), lambda i: (0, 0)),
        pl.BlockSpec((1, W), lambda i: (0, 0)),
        pl.BlockSpec((1, 1), lambda i: (0, 0)),
    ],
    out_specs=pl.BlockSpec((BDE // 4, 4), lambda i: (i, 0)),
    out_shape=jax.ShapeDtypeStruct((EP // 4, 4), jnp.float32),
)


def _combine_body(slf_ref, agg_ref, m_ref, seg_ref, ws_ref, out_ref):
    inv = 1.0 / (seg_ref[...] + 1e-9)
    wsum = ws_ref[0] + ws_ref[1]
    agg = jnp.concatenate([agg_ref[0], agg_ref[1]], axis=1)
    mm = jnp.concatenate([m_ref[0], m_ref[1]], axis=1)
    out_ref[...] = jnp.tanh(slf_ref[...] + agg * inv + wsum * mm)


_combine = pl.pallas_call(
    _combine_body,
    grid=(NP // BD,),
    in_specs=[
        pl.BlockSpec((BD, D), lambda i: (i, 0)),
        pl.BlockSpec((2, BD, 32), lambda i: (0, i, 0)),
        pl.BlockSpec((2, BD, 32), lambda i: (0, i, 0)),
        pl.BlockSpec((BD, 1), lambda i: (i, 0)),
        pl.BlockSpec((2, BD, 1), lambda i: (0, i, 0)),
    ],
    out_specs=pl.BlockSpec((BD, D), lambda i: (i, 0)),
    out_shape=jax.ShapeDtypeStruct((NP, D), jnp.float32),
)


@functools.partial(
    pl.kernel,
    out_type=jax.ShapeDtypeStruct((2, EP, 32), jnp.float32),
    mesh=_SC_MESH,
    compiler_params=pltpu.CompilerParams(needs_layout_passes=False, use_tc_tiling_on_sc=False),
    scratch_types=[
        pltpu.VMEM((IC1, W), jnp.int32),
        pltpu.VMEM((IC1, W), jnp.int32),
        pltpu.VMEM((B1 * W, 32), jnp.float32),
        pltpu.VMEM((B1 * W, 32), jnp.float32),
        pltpu.VMEM_SHARED((NP, 32), jnp.float32),
        pltpu.VMEM_SHARED((DSTN, 32), jnp.float32),
        pltpu.SemaphoreType.DMA,
    ],
)
def _phase1(src_hbm, dst_hbm, asrc_hbm, adst_hbm, s_hbm,
            srcc, dstc, bufs, bufd,
            asrc_sp, adst_sp, sem):
    cid = lax.axis_index("c")
    sid = lax.axis_index("s")
    pltpu.sync_copy(asrc_hbm.at[cid, pl.ds(sid * _STG, _STG)],
                    asrc_sp.at[pl.ds(sid * _STG, _STG)])
    _dstg = DSTN // 16
    pltpu.sync_copy(adst_hbm.at[cid, pl.ds(DSTLO + sid * _dstg, _dstg)],
                    adst_sp.at[pl.ds(sid * _dstg, _dstg)])
    plsc.subcore_barrier()
    rows_per_tile = ROWS // 16
    row0 = sid * rows_per_tile

    def chunk_body(t, carry):
        base = row0 + t * IC1
        pltpu.sync_copy(src_hbm.at[pl.ds(base, IC1)], srcc)
        pltpu.sync_copy(dst_hbm.at[pl.ds(base, IC1)], dstc)
        for r2 in range(IC1):
            for g in range(W // 16):
                dstc[r2, pl.ds(g * 16, 16)] = (
                    dstc[r2, pl.ds(g * 16, 16)] - DSTLO)
        for u in range(IC1 // B1):
            hs = []
            for k in range(B1):
                hs.append(pltpu.async_copy(
                    asrc_sp.at[srcc.at[u * B1 + k]],
                    bufs.at[pl.ds(k * W, W)], sem))
                hs.append(pltpu.async_copy(
                    adst_sp.at[dstc.at[u * B1 + k]],
                    bufd.at[pl.ds(k * W, W)], sem))
            for h in hs:
                h.wait()

            def rcomp(r2, c):
                def gcomp(g, c2):
                    e0 = r2 * W + g * 16
                    for j in range(16):
                        e = e0 + j
                        bufs[e, pl.ds(0, 16)] = (
                            bufs[e, pl.ds(0, 16)] + bufd[e, pl.ds(0, 16)])
                        bufs[e, pl.ds(16, 16)] = (
                            bufs[e, pl.ds(16, 16)] + bufd[e, pl.ds(16, 16)])
                    return c2

                lax.fori_loop(0, W // 16, gcomp, 0)
                return c

            lax.fori_loop(0, B1, rcomp, 0)
            pltpu.sync_copy(
                bufs, s_hbm.at[cid, pl.ds((base + u * B1) * W, B1 * W)])
        return carry

    lax.fori_loop(0, rows_per_tile // IC1, chunk_body, 0)


@functools.partial(
    pl.kernel,
    out_type=(jax.ShapeDtypeStruct((2, NP, 32), jnp.float32),
              jax.ShapeDtypeStruct((NP,), jnp.float32)),
    mesh=_SC_MESH,
    compiler_params=pltpu.CompilerParams(needs_layout_passes=False, use_tc_tiling_on_sc=False),
    scratch_types=[
        pltpu.VMEM((B2, W), jnp.int32),
        pltpu.VMEM((B2, W), jnp.int32),
        pltpu.VMEM((B2, W), jnp.int32),
        pltpu.VMEM((B2, W), jnp.float32),
        pltpu.VMEM((B2 * W, 32), jnp.float32),
        pltpu.VMEM_SHARED((NP, 32), jnp.float32),
        pltpu.VMEM_SHARED((NP,), jnp.float32),
        pltpu.VMEM_SHARED((DSTN, 32), jnp.float32),
        pltpu.SemaphoreType.DMA,
        pltpu.SemaphoreType.DMA,
    ],
)
def _phase2(src_hbm, dst_hbm, ev_hbm, m_hbm, zagg_hbm, zseg_hbm,
            agg_out, seg_out, srcm, dstm, dlocm, evm, mbuf,
            agg_sp, seg_sp, m_sp, sem, sem2):
    cid = lax.axis_index("c")
    sid = lax.axis_index("s")
    pltpu.sync_copy(zagg_hbm.at[pl.ds(sid * _STG, _STG)],
                    agg_sp.at[pl.ds(sid * _STG, _STG)])

    @pl.when(cid == 0)
    def _():
        pltpu.sync_copy(zseg_hbm.at[pl.ds(sid * _STG, _STG)],
                        seg_sp.at[pl.ds(sid * _STG, _STG)])

    _dstg = DSTN // 16
    pltpu.sync_copy(m_hbm.at[cid, pl.ds(DSTLO + sid * _dstg, _dstg)],
                    m_sp.at[pl.ds(sid * _dstg, _dstg)])
    plsc.subcore_barrier()
    rows_per_tile = ROWS // 16
    row0 = sid * rows_per_tile

    def it_body(t, carry):
        base = row0 + t * B2
        pltpu.sync_copy(src_hbm.at[pl.ds(base, B2)], srcm)
        pltpu.sync_copy(dst_hbm.at[pl.ds(base, B2)], dstm)
        pltpu.sync_copy(ev_hbm.at[pl.ds(base, B2)], evm)
        for r2 in range(B2):
            for g in range(W // 16):
                dlocm[r2, pl.ds(g * 16, 16)] = (
                    dstm[r2, pl.ds(g * 16, 16)] - DSTLO)
        hs = []
        for k in range(B2):
            hs.append(pltpu.async_copy(
                m_sp.at[dlocm.at[k]], mbuf.at[pl.ds(k * W, W)], sem))
        for h in hs:
            h.wait()

        def rcomp(r2, c):
            def gcomp(g, c2):
                evec = evm[r2, pl.ds(g * 16, 16)]
                e0 = r2 * W + g * 16
                for j in range(16):
                    e = e0 + j
                    evs = evec[j]
                    mbuf[e, pl.ds(0, 16)] = mbuf[e, pl.ds(0, 16)] * evs
                    mbuf[e, pl.ds(16, 16)] = mbuf[e, pl.ds(16, 16)] * evs
                return c2

            lax.fori_loop(0, W // 16, gcomp, 0)
            return c

        lax.fori_loop(0, B2, rcomp, 0)
        hs2 = []
        for k in range(B2):
            hs2.append(pltpu.async_copy(
                mbuf.at[pl.ds(k * W, W)], agg_sp.at[srcm.at[k]], sem2,
                add=True))

        @pl.when(cid == 0)
        def _():
            evhs = []
            for k in range(B2):
                evhs.append(pltpu.async_copy(
                    evm.at[k], seg_sp.at[srcm.at[k]], sem2, add=True))
            for h in evhs:
                h.wait()

        for h in hs2:
            h.wait()
        return carry

    lax.fori_loop(0, rows_per_tile // B2, it_body, 0)
    plsc.subcore_barrier()
    pltpu.sync_copy(agg_sp.at[pl.ds(sid * _STG, _STG)],
                    agg_out.at[cid, pl.ds(sid * _STG, _STG)])

    @pl.when(cid == 0)
    def _():
        pltpu.sync_copy(seg_sp.at[pl.ds(sid * _STG, _STG)],
                        seg_out.at[pl.ds(sid * _STG, _STG)])


@functools.partial(
    pl.kernel,
    out_type=jax.ShapeDtypeStruct((2, NP), jnp.float32),
    mesh=_SC_MESH,
    compiler_params=pltpu.CompilerParams(needs_layout_passes=False, use_tc_tiling_on_sc=False),
    scratch_types=[
        pltpu.VMEM((BW, W), jnp.int32),
        pltpu.VMEM((BW, W), jnp.float32),
        pltpu.VMEM_SHARED((NP,), jnp.float32),
        pltpu.SemaphoreType.DMA,
    ],
)
def _wsum(src_hbm, w_hbm, zseg_hbm, out_hbm, srcm, wm, seg_sp, sem):
    cid = lax.axis_index("c")
    sid = lax.axis_index("s")
    pltpu.sync_copy(zseg_hbm.at[pl.ds(sid * _STG, _STG)],
                    seg_sp.at[pl.ds(sid * _STG, _STG)])
    plsc.subcore_barrier()
    wid = cid * 16 + sid
    rows_per_tile = EROWS // 32
    row0 = wid * rows_per_tile

    def it_body(t, carry):
        base = row0 + t * BW
        pltpu.sync_copy(src_hbm.at[pl.ds(base, BW)], srcm)
        pltpu.sync_copy(w_hbm.at[pl.ds(base, BW)], wm)
        hs = []
        for k in range(BW):
            hs.append(pltpu.async_copy(
                wm.at[k], seg_sp.at[srcm.at[k]], sem, add=True))
        for h in hs:
            h.wait()
        return carry

    lax.fori_loop(0, rows_per_tile // BW, it_body, 0)
    plsc.subcore_barrier()
    pltpu.sync_copy(seg_sp.at[pl.ds(sid * _STG, _STG)],
                    out_hbm.at[cid, pl.ds(sid * _STG, _STG)])


def _pad_idx(n):
    return PADBASE + (jnp.arange(n, dtype=jnp.int32) % 64)


def _view(emb, er_src, er_dst, ee_src, ee_w, Wa, ba, w0, b0, Ws, bs, Wn, bn):
    x = jnp.pad(emb, ((0, NP - N), (0, 0)))
    src2d = jnp.concatenate([er_src.astype(jnp.int32),
                             _pad_idx(EP - E_ER)]).reshape(ROWS, W)
    dst2d = jnp.concatenate([er_dst.astype(jnp.int32),
                             _pad_idx(EP - E_ER)]).reshape(ROWS, W)
    esrc2d = jnp.concatenate([ee_src.astype(jnp.int32),
                              _pad_idx(EEP - E_EE)]).reshape(EROWS, W)
    ew2d = jnp.concatenate([ee_w, jnp.zeros(EEP - E_EE, jnp.float32)]
                           ).reshape(EROWS, W)
    zagg = jnp.zeros((NP, 32), jnp.float32)
    zseg = jnp.zeros((NP,), jnp.float32)
    ws3 = _wsum(esrc2d, ew2d, zseg).reshape(2, NP, 1)
    for l in range(L):
        W4 = jnp.concatenate([Wa[l][D:], Wa[l][:D], Wn[l], Ws[l]], axis=1)
        b4 = jnp.concatenate([ba[l], jnp.zeros(D, jnp.float32), bn[l], bs[l]]
                             ).reshape(1, 4 * D)
        asrc, adst, m01, slf = _dense(x, W4, b4)
        s01 = _phase1(src2d, dst2d, asrc, adst)
        w0f = w0[l].reshape(D)
        w0a4 = jnp.tile(w0f[0:32], 4).reshape(1, W)
        w0b4 = jnp.tile(w0f[32:64], 4).reshape(1, W)
        ev2d = _ev(s01.reshape(2, EP // 4, W), w0a4, w0b4,
                   b0[l].reshape(1, 1)).reshape(ROWS, W)
        agg, seg = _phase2(src2d, dst2d, ev2d, m01, zagg, zseg)
        x = _combine(slf, agg, m01, seg.reshape(NP, 1), ws3)
    return x


def kernel(er_src_H, er_dst_H, ee_src_H, ee_weight_H,
           er_src_T, er_dst_T, ee_src_T, ee_weight_T,
           embH, embT,
           WH_attn, bH_attn, wH_0, bH_0, WH_self, bH_self, WH_neigh, bH_neigh,
           WT_attn, bT_attn, wT_0, bT_0, WT_self, bT_self, WT_neigh, bT_neigh):
    xH = _view(embH, er_src_H, er_dst_H, ee_src_H, ee_weight_H,
               WH_attn, bH_attn, wH_0, bH_0, WH_self, bH_self,
               WH_neigh, bH_neigh)
    xT = _view(embT, er_src_T, er_dst_T, ee_src_T, ee_weight_T,
               WT_attn, bT_attn, wT_0, bT_0, WT_self, bT_self,
               WT_neigh, bT_neigh)
    return (xH[:NE], xH[NE:N], xT[:NE], xT[NE:N])


# speedup vs baseline: 12.8933x; 1.0400x over previous
import functools

import jax
import jax.numpy as jnp
from jax import lax
from jax.experimental import pallas as pl
from jax.experimental.pallas import tpu as pltpu
from jax.experimental.pallas import tpu_sc as plsc

NE = 50000
NR = 1000
N = NE + NR
D = 64
L = 2
NP = 51200
E_ER = 800000
E_EE = 200000
W = 128
ROWS = 6400
EROWS = 1600
EP = ROWS * W
EEP = EROWS * W
DSTLO = NE
DSTN = 1088
PADBASE = 51008
BD = 1024

_SC_MESH = plsc.VectorSubcoreMesh(core_axis_name="c", subcore_axis_name="s")
_STG = NP // 16
B1 = 2
IC1 = 8
B2 = 4
IC2 = 8
BW = 10


def _dense_body(x_ref, w_ref, b_ref, asrc_ref, adst_ref, m_ref, slf_ref):
    y = jnp.dot(x_ref[...], w_ref[...], preferred_element_type=jnp.float32)
    y = y + b_ref[...]
    asrc_ref[...] = jnp.stack([y[:, 0:32], y[:, 32:64]])
    adst_ref[...] = jnp.stack([y[:, 64:96], y[:, 96:128]])
    m_ref[...] = jnp.stack([y[:, 128:160], y[:, 160:192]])
    slf_ref[...] = y[:, 192:256]


_dense = pl.pallas_call(
    _dense_body,
    grid=(NP // BD,),
    in_specs=[
        pl.BlockSpec((BD, D), lambda i: (i, 0)),
        pl.BlockSpec((D, 4 * D), lambda i: (0, 0)),
        pl.BlockSpec((1, 4 * D), lambda i: (0, 0)),
    ],
    out_specs=[
        pl.BlockSpec((2, BD, 32), lambda i: (0, i, 0)),
        pl.BlockSpec((2, BD, 32), lambda i: (0, i, 0)),
        pl.BlockSpec((2, BD, 32), lambda i: (0, i, 0)),
        pl.BlockSpec((BD, D), lambda i: (i, 0)),
    ],
    out_shape=[
        jax.ShapeDtypeStruct((2, NP, 32), jnp.float32),
        jax.ShapeDtypeStruct((2, NP, 32), jnp.float32),
        jax.ShapeDtypeStruct((2, NP, 32), jnp.float32),
        jax.ShapeDtypeStruct((NP, D), jnp.float32),
    ],
)


BDE = 8192


def _ev_body(s_ref, w0a_ref, w0b_ref, b0_ref, ev_ref):
    q = (jnp.tanh(s_ref[0]) * w0a_ref[...]
         + jnp.tanh(s_ref[1]) * w0b_ref[...])
    lane = lax.broadcasted_iota(jnp.int32, (W, 4), 0)
    col = lax.broadcasted_iota(jnp.int32, (W, 4), 1)
    sel = jnp.where(lane // 32 == col, 1.0, 0.0)
    logit4 = jnp.dot(q, sel, preferred_element_type=jnp.float32)
    ev_ref[...] = jnp.exp(logit4 + b0_ref[0, 0])


_ev = pl.pallas_call(
    _ev_body,
    grid=(EP // BDE,),
    in_specs=[
        pl.BlockSpec((2, BDE // 4, W), lambda i: (0, i, 0)),
        pl.BlockSpec((1, W), lambda i: (0, 0)),
        pl.BlockSpec((1, W), lambda i: (0, 0)),
        pl.BlockSpec((1, 1), lambda i: (0, 0)),
    ],
    out_specs=pl.BlockSpec((BDE // 4, 4), lambda i: (i, 0)),
    out_shape=jax.ShapeDtypeStruct((EP // 4, 4), jnp.float32),
)


def _combine_body(slf_ref, agg_ref, m_ref, seg_ref, ws_ref, out_ref):
    inv = 1.0 / (seg_ref[...] + 1e-9)
    wsum = ws_ref[0] + ws_ref[1]
    agg = jnp.concatenate([agg_ref[0], agg_ref[1]], axis=1)
    mm = jnp.concatenate([m_ref[0], m_ref[1]], axis=1)
    out_ref[...] = jnp.tanh(slf_ref[...] + agg * inv + wsum * mm)


_combine = pl.pallas_call(
    _combine_body,
    grid=(NP // BD,),
    in_specs=[
        pl.BlockSpec((BD, D), lambda i: (i, 0)),
        pl.BlockSpec((2, BD, 32), lambda i: (0, i, 0)),
        pl.BlockSpec((2, BD, 32), lambda i: (0, i, 0)),
        pl.BlockSpec((BD, 1), lambda i: (i, 0)),
        pl.BlockSpec((2, BD, 1), lambda i: (0, i, 0)),
    ],
    out_specs=pl.BlockSpec((BD, D), lambda i: (i, 0)),
    out_shape=jax.ShapeDtypeStruct((NP, D), jnp.float32),
)


@functools.partial(
    pl.kernel,
    out_type=jax.ShapeDtypeStruct((2, EP, 32), jnp.float32),
    mesh=_SC_MESH,
    compiler_params=pltpu.CompilerParams(needs_layout_passes=False, use_tc_tiling_on_sc=False),
    scratch_types=[
        pltpu.VMEM((IC1, W), jnp.int32),
        pltpu.VMEM((IC1, W), jnp.int32),
        pltpu.VMEM((B1 * W, 32), jnp.float32),
        pltpu.VMEM((B1 * W, 32), jnp.float32),
        pltpu.VMEM_SHARED((NP, 32), jnp.float32),
        pltpu.VMEM_SHARED((DSTN, 32), jnp.float32),
        pltpu.SemaphoreType.DMA,
    ],
)
def _phase1(src_hbm, dst_hbm, asrc_hbm, adst_hbm, s_hbm,
            srcc, dstc, bufs, bufd,
            asrc_sp, adst_sp, sem):
    cid = lax.axis_index("c")
    sid = lax.axis_index("s")
    pltpu.sync_copy(asrc_hbm.at[cid, pl.ds(sid * _STG, _STG)],
                    asrc_sp.at[pl.ds(sid * _STG, _STG)])
    _dstg = DSTN // 16
    pltpu.sync_copy(adst_hbm.at[cid, pl.ds(DSTLO + sid * _dstg, _dstg)],
                    adst_sp.at[pl.ds(sid * _dstg, _dstg)])
    plsc.subcore_barrier()
    rows_per_tile = ROWS // 16
    row0 = sid * rows_per_tile

    def chunk_body(t, carry):
        base = row0 + t * IC1
        pltpu.sync_copy(src_hbm.at[pl.ds(base, IC1)], srcc)
        pltpu.sync_copy(dst_hbm.at[pl.ds(base, IC1)], dstc)
        for r2 in range(IC1):
            for g in range(W // 16):
                dstc[r2, pl.ds(g * 16, 16)] = (
                    dstc[r2, pl.ds(g * 16, 16)] - DSTLO)
        for u in range(IC1 // B1):
            hs = []
            for k in range(B1):
                hs.append(pltpu.async_copy(
                    asrc_sp.at[srcc.at[u * B1 + k]],
                    bufs.at[pl.ds(k * W, W)], sem))
                hs.append(pltpu.async_copy(
                    adst_sp.at[dstc.at[u * B1 + k]],
                    bufd.at[pl.ds(k * W, W)], sem))
            for h in hs:
                h.wait()

            def rcomp(r2, c):
                def gcomp(g, c2):
                    e0 = r2 * W + g * 16
                    for j in range(16):
                        e = e0 + j
                        bufs[e, pl.ds(0, 16)] = (
                            bufs[e, pl.ds(0, 16)] + bufd[e, pl.ds(0, 16)])
                        bufs[e, pl.ds(16, 16)] = (
                            bufs[e, pl.ds(16, 16)] + bufd[e, pl.ds(16, 16)])
                    return c2

                lax.fori_loop(0, W // 16, gcomp, 0)
                return c

            lax.fori_loop(0, B1, rcomp, 0)
            pltpu.sync_copy(
                bufs, s_hbm.at[cid, pl.ds((base + u * B1) * W, B1 * W)])
        return carry

    lax.fori_loop(0, rows_per_tile // IC1, chunk_body, 0)


@functools.partial(
    pl.kernel,
    out_type=(jax.ShapeDtypeStruct((2, NP, 32), jnp.float32),
              jax.ShapeDtypeStruct((NP,), jnp.float32)),
    mesh=_SC_MESH,
    compiler_params=pltpu.CompilerParams(needs_layout_passes=False, use_tc_tiling_on_sc=False),
    scratch_types=[
        pltpu.VMEM((IC2, W), jnp.int32),
        pltpu.VMEM((IC2, W), jnp.int32),
        pltpu.VMEM((IC2, W), jnp.float32),
        pltpu.VMEM((B2 * W, 32), jnp.float32),
        pltpu.VMEM_SHARED((NP, 32), jnp.float32),
        pltpu.VMEM_SHARED((NP,), jnp.float32),
        pltpu.VMEM_SHARED((DSTN, 32), jnp.float32),
        pltpu.SemaphoreType.DMA,
        pltpu.SemaphoreType.DMA,
    ],
)
def _phase2(src_hbm, dst_hbm, ev_hbm, m_hbm, zagg_hbm, zseg_hbm,
            agg_out, seg_out, srcc, dstc, evc, mbuf,
            agg_sp, seg_sp, m_sp, sem, sem2):
    cid = lax.axis_index("c")
    sid = lax.axis_index("s")
    pltpu.sync_copy(zagg_hbm.at[pl.ds(sid * _STG, _STG)],
                    agg_sp.at[pl.ds(sid * _STG, _STG)])

    @pl.when(cid == 0)
    def _():
        pltpu.sync_copy(zseg_hbm.at[pl.ds(sid * _STG, _STG)],
                        seg_sp.at[pl.ds(sid * _STG, _STG)])

    _dstg = DSTN // 16
    pltpu.sync_copy(m_hbm.at[cid, pl.ds(DSTLO + sid * _dstg, _dstg)],
                    m_sp.at[pl.ds(sid * _dstg, _dstg)])
    plsc.subcore_barrier()
    rows_per_tile = ROWS // 16
    row0 = sid * rows_per_tile

    def chunk_body(t, carry):
        base = row0 + t * IC2
        pltpu.sync_copy(src_hbm.at[pl.ds(base, IC2)], srcc)
        pltpu.sync_copy(dst_hbm.at[pl.ds(base, IC2)], dstc)
        pltpu.sync_copy(ev_hbm.at[pl.ds(base, IC2)], evc)
        for r2 in range(IC2):
            for g in range(W // 16):
                dstc[r2, pl.ds(g * 16, 16)] = (
                    dstc[r2, pl.ds(g * 16, 16)] - DSTLO)
        for u in range(IC2 // B2):
            hs = []
            for k in range(B2):
                hs.append(pltpu.async_copy(
                    m_sp.at[dstc.at[u * B2 + k]],
                    mbuf.at[pl.ds(k * W, W)], sem))
            for h in hs:
                h.wait()

            def rcomp(r2, c):
                def gcomp(g, c2):
                    evec = evc[u * B2 + r2, pl.ds(g * 16, 16)]
                    e0 = r2 * W + g * 16
                    for j in range(16):
                        e = e0 + j
                        evs = evec[j]
                        mbuf[e, pl.ds(0, 16)] = mbuf[e, pl.ds(0, 16)] * evs
                        mbuf[e, pl.ds(16, 16)] = mbuf[e, pl.ds(16, 16)] * evs
                    return c2

                lax.fori_loop(0, W // 16, gcomp, 0)
                return c

            lax.fori_loop(0, B2, rcomp, 0)
            hs2 = []
            for k in range(B2):
                hs2.append(pltpu.async_copy(
                    mbuf.at[pl.ds(k * W, W)], agg_sp.at[srcc.at[u * B2 + k]],
                    sem2, add=True))

            @pl.when(cid == 0)
            def _():
                evhs = []
                for k in range(B2):
                    evhs.append(pltpu.async_copy(
                        evc.at[u * B2 + k], seg_sp.at[srcc.at[u * B2 + k]],
                        sem2, add=True))
                for h in evhs:
                    h.wait()

            for h in hs2:
                h.wait()
        return carry

    lax.fori_loop(0, rows_per_tile // IC2, chunk_body, 0)
    plsc.subcore_barrier()
    pltpu.sync_copy(agg_sp.at[pl.ds(sid * _STG, _STG)],
                    agg_out.at[cid, pl.ds(sid * _STG, _STG)])

    @pl.when(cid == 0)
    def _():
        pltpu.sync_copy(seg_sp.at[pl.ds(sid * _STG, _STG)],
                        seg_out.at[pl.ds(sid * _STG, _STG)])


@functools.partial(
    pl.kernel,
    out_type=jax.ShapeDtypeStruct((2, NP), jnp.float32),
    mesh=_SC_MESH,
    compiler_params=pltpu.CompilerParams(needs_layout_passes=False, use_tc_tiling_on_sc=False),
    scratch_types=[
        pltpu.VMEM((BW, W), jnp.int32),
        pltpu.VMEM((BW, W), jnp.float32),
        pltpu.VMEM_SHARED((NP,), jnp.float32),
        pltpu.SemaphoreType.DMA,
    ],
)
def _wsum(src_hbm, w_hbm, zseg_hbm, out_hbm, srcm, wm, seg_sp, sem):
    cid = lax.axis_index("c")
    sid = lax.axis_index("s")
    pltpu.sync_copy(zseg_hbm.at[pl.ds(sid * _STG, _STG)],
                    seg_sp.at[pl.ds(sid * _STG, _STG)])
    plsc.subcore_barrier()
    wid = cid * 16 + sid
    rows_per_tile = EROWS // 32
    row0 = wid * rows_per_tile

    def it_body(t, carry):
        base = row0 + t * BW
        pltpu.sync_copy(src_hbm.at[pl.ds(base, BW)], srcm)
        pltpu.sync_copy(w_hbm.at[pl.ds(base, BW)], wm)
        hs = []
        for k in range(BW):
            hs.append(pltpu.async_copy(
                wm.at[k], seg_sp.at[srcm.at[k]], sem, add=True))
        for h in hs:
            h.wait()
        return carry

    lax.fori_loop(0, rows_per_tile // BW, it_body, 0)
    plsc.subcore_barrier()
    pltpu.sync_copy(seg_sp.at[pl.ds(sid * _STG, _STG)],
                    out_hbm.at[cid, pl.ds(sid * _STG, _STG)])


def _pad_idx(n):
    return PADBASE + (jnp.arange(n, dtype=jnp.int32) % 64)


def _view(emb, er_src, er_dst, ee_src, ee_w, Wa, ba, w0, b0, Ws, bs, Wn, bn):
    x = jnp.pad(emb, ((0, NP - N), (0, 0)))
    src2d = jnp.concatenate([er_src.astype(jnp.int32),
                             _pad_idx(EP - E_ER)]).reshape(ROWS, W)
    dst2d = jnp.concatenate([er_dst.astype(jnp.int32),
                             _pad_idx(EP - E_ER)]).reshape(ROWS, W)
    esrc2d = jnp.concatenate([ee_src.astype(jnp.int32),
                              _pad_idx(EEP - E_EE)]).reshape(EROWS, W)
    ew2d = jnp.concatenate([ee_w, jnp.zeros(EEP - E_EE, jnp.float32)]
                           ).reshape(EROWS, W)
    zagg = jnp.zeros((NP, 32), jnp.float32)
    zseg = jnp.zeros((NP,), jnp.float32)
    ws3 = _wsum(esrc2d, ew2d, zseg).reshape(2, NP, 1)
    for l in range(L):
        W4 = jnp.concatenate([Wa[l][D:], Wa[l][:D], Wn[l], Ws[l]], axis=1)
        b4 = jnp.concatenate([ba[l], jnp.zeros(D, jnp.float32), bn[l], bs[l]]
                             ).reshape(1, 4 * D)
        asrc, adst, m01, slf = _dense(x, W4, b4)
        s01 = _phase1(src2d, dst2d, asrc, adst)
        w0f = w0[l].reshape(D)
        w0a4 = jnp.tile(w0f[0:32], 4).reshape(1, W)
        w0b4 = jnp.tile(w0f[32:64], 4).reshape(1, W)
        ev2d = _ev(s01.reshape(2, EP // 4, W), w0a4, w0b4,
                   b0[l].reshape(1, 1)).reshape(ROWS, W)
        agg, seg = _phase2(src2d, dst2d, ev2d, m01, zagg, zseg)
        x = _combine(slf, agg, m01, seg.reshape(NP, 1), ws3)
    return x


def kernel(er_src_H, er_dst_H, ee_src_H, ee_weight_H,
           er_src_T, er_dst_T, ee_src_T, ee_weight_T,
           embH, embT,
           WH_attn, bH_attn, wH_0, bH_0, WH_self, bH_self, WH_neigh, bH_neigh,
           WT_attn, bT_attn, wT_0, bT_0, WT_self, bT_self, WT_neigh, bT_neigh):
    xH = _view(embH, er_src_H, er_dst_H, ee_src_H, ee_weight_H,
               WH_attn, bH_attn, wH_0, bH_0, WH_self, bH_self,
               WH_neigh, bH_neigh)
    xT = _view(embT, er_src_T, er_dst_T, ee_src_T, ee_weight_T,
               WT_attn, bT_attn, wT_0, bT_0, WT_self, bT_self,
               WT_neigh, bT_neigh)
    return (xH[:NE], xH[NE:N], xT[:NE], xT[NE:N])


# speedup vs baseline: 20.8596x; 1.6179x over previous
import functools

import jax
import jax.numpy as jnp
from jax import lax
from jax.experimental import pallas as pl
from jax.experimental.pallas import tpu as pltpu
from jax.experimental.pallas import tpu_sc as plsc

NE = 50000
NR = 1000
N = NE + NR
D = 64
L = 2
NP = 51200
E_ER = 800000
E_EE = 200000
W = 128
ROWS = 6400
EROWS = 1600
EP = ROWS * W
EEP = EROWS * W
DSTLO = NE
DSTN = 1088
PADBASE = 51008
BD = 1024

_SC_MESH = plsc.VectorSubcoreMesh(core_axis_name="c", subcore_axis_name="s")
_STG = NP // 16
B1 = 5
IC1 = 10
B2 = 4
IC2 = 8
BW = 10


def _dense_body(x_ref, w_ref, b_ref, asrc_ref, adst_ref, m_ref, slf_ref):
    y = jnp.dot(x_ref[...], w_ref[...], preferred_element_type=jnp.float32)
    y = y + b_ref[...]
    asrc_ref[...] = jnp.stack([y[:, 0:32], y[:, 32:64]])
    adst_ref[...] = jnp.stack([y[:, 64:96], y[:, 96:128]])
    m_ref[...] = jnp.stack([y[:, 128:160], y[:, 160:192]])
    slf_ref[...] = y[:, 192:256]


_dense = pl.pallas_call(
    _dense_body,
    grid=(NP // BD,),
    in_specs=[
        pl.BlockSpec((BD, D), lambda i: (i, 0)),
        pl.BlockSpec((D, 4 * D), lambda i: (0, 0)),
        pl.BlockSpec((1, 4 * D), lambda i: (0, 0)),
    ],
    out_specs=[
        pl.BlockSpec((2, BD, 32), lambda i: (0, i, 0)),
        pl.BlockSpec((2, BD, 32), lambda i: (0, i, 0)),
        pl.BlockSpec((2, BD, 32), lambda i: (0, i, 0)),
        pl.BlockSpec((BD, D), lambda i: (i, 0)),
    ],
    out_shape=[
        jax.ShapeDtypeStruct((2, NP, 32), jnp.float32),
        jax.ShapeDtypeStruct((2, NP, 32), jnp.float32),
        jax.ShapeDtypeStruct((2, NP, 32), jnp.float32),
        jax.ShapeDtypeStruct((NP, D), jnp.float32),
    ],
)


BDE = 8192


def _ev_body(s_ref, w0a_ref, w0b_ref, b0_ref, ev_ref):
    q = (jnp.tanh(s_ref[0]) * w0a_ref[...]
         + jnp.tanh(s_ref[1]) * w0b_ref[...])
    lane = lax.broadcasted_iota(jnp.int32, (W, 4), 0)
    col = lax.broadcasted_iota(jnp.int32, (W, 4), 1)
    sel = jnp.where(lane // 32 == col, 1.0, 0.0)
    logit4 = jnp.dot(q, sel, preferred_element_type=jnp.float32)
    ev_ref[...] = jnp.exp(logit4 + b0_ref[0, 0])


_ev = pl.pallas_call(
    _ev_body,
    grid=(EP // BDE,),
    in_specs=[
        pl.BlockSpec((2, BDE // 4, W), lambda i: (0, i, 0)),
        pl.BlockSpec((1, W), lambda i: (0, 0)),
        pl.BlockSpec((1, W), lambda i: (0, 0)),
        pl.BlockSpec((1, 1), lambda i: (0, 0)),
    ],
    out_specs=pl.BlockSpec((BDE // 4, 4), lambda i: (i, 0)),
    out_shape=jax.ShapeDtypeStruct((EP // 4, 4), jnp.float32),
)


def _combine_body(slf_ref, agg_ref, m_ref, seg_ref, ws_ref, out_ref):
    inv = 1.0 / (seg_ref[...] + 1e-9)
    wsum = ws_ref[0] + ws_ref[1]
    agg = jnp.concatenate([agg_ref[0], agg_ref[1]], axis=1)
    mm = jnp.concatenate([m_ref[0], m_ref[1]], axis=1)
    out_ref[...] = jnp.tanh(slf_ref[...] + agg * inv + wsum * mm)


_combine = pl.pallas_call(
    _combine_body,
    grid=(NP // BD,),
    in_specs=[
        pl.BlockSpec((BD, D), lambda i: (i, 0)),
        pl.BlockSpec((2, BD, 32), lambda i: (0, i, 0)),
        pl.BlockSpec((2, BD, 32), lambda i: (0, i, 0)),
        pl.BlockSpec((BD, 1), lambda i: (i, 0)),
        pl.BlockSpec((2, BD, 1), lambda i: (0, i, 0)),
    ],
    out_specs=pl.BlockSpec((BD, D), lambda i: (i, 0)),
    out_shape=jax.ShapeDtypeStruct((NP, D), jnp.float32),
)


@functools.partial(
    pl.kernel,
    out_type=jax.ShapeDtypeStruct((2, EP, 32), jnp.float32),
    mesh=_SC_MESH,
    compiler_params=pltpu.CompilerParams(needs_layout_passes=False, use_tc_tiling_on_sc=False),
    scratch_types=[
        pltpu.VMEM((IC1, W), jnp.int32),
        pltpu.VMEM((IC1, W), jnp.int32),
        pltpu.VMEM((B1 * W, 32), jnp.float32),
        pltpu.VMEM_SHARED((NP, 32), jnp.float32),
        pltpu.VMEM_SHARED((DSTN, 32), jnp.float32),
        pltpu.SemaphoreType.DMA,
    ],
)
def _phase1(src_hbm, dst_hbm, asrc_hbm, adst_hbm, s_hbm,
            srcc, dstc, bufs,
            asrc_sp, adst_sp, sem):
    cid = lax.axis_index("c")
    sid = lax.axis_index("s")
    pltpu.sync_copy(asrc_hbm.at[cid, pl.ds(sid * _STG, _STG)],
                    asrc_sp.at[pl.ds(sid * _STG, _STG)])
    _dstg = DSTN // 16
    pltpu.sync_copy(adst_hbm.at[cid, pl.ds(DSTLO + sid * _dstg, _dstg)],
                    adst_sp.at[pl.ds(sid * _dstg, _dstg)])
    plsc.subcore_barrier()
    rows_per_tile = ROWS // 16
    row0 = sid * rows_per_tile

    def chunk_body(t, carry):
        base = row0 + t * IC1
        pltpu.sync_copy(src_hbm.at[pl.ds(base, IC1)], srcc)
        pltpu.sync_copy(dst_hbm.at[pl.ds(base, IC1)], dstc)
        for r2 in range(IC1):
            for g in range(W // 16):
                dstc[r2, pl.ds(g * 16, 16)] = (
                    dstc[r2, pl.ds(g * 16, 16)] - DSTLO)
        for u in range(IC1 // B1):
            hs = []
            for k in range(B1):
                hs.append(pltpu.async_copy(
                    asrc_sp.at[srcc.at[u * B1 + k]],
                    bufs.at[pl.ds(k * W, W)], sem))
            for h in hs:
                h.wait()
            hs2 = []
            for k in range(B1):
                hs2.append(pltpu.async_copy(
                    adst_sp.at[dstc.at[u * B1 + k]],
                    bufs.at[pl.ds(k * W, W)], sem, add=True))
            for h in hs2:
                h.wait()
            pltpu.sync_copy(
                bufs, s_hbm.at[cid, pl.ds((base + u * B1) * W, B1 * W)])
        return carry

    lax.fori_loop(0, rows_per_tile // IC1, chunk_body, 0)


@functools.partial(
    pl.kernel,
    out_type=(jax.ShapeDtypeStruct((2, NP, 32), jnp.float32),
              jax.ShapeDtypeStruct((NP,), jnp.float32)),
    mesh=_SC_MESH,
    compiler_params=pltpu.CompilerParams(needs_layout_passes=False, use_tc_tiling_on_sc=False),
    scratch_types=[
        pltpu.VMEM((IC2, W), jnp.int32),
        pltpu.VMEM((IC2, W), jnp.int32),
        pltpu.VMEM((IC2, W), jnp.float32),
        pltpu.VMEM((B2 * W, 32), jnp.float32),
        pltpu.VMEM_SHARED((NP, 32), jnp.float32),
        pltpu.VMEM_SHARED((NP,), jnp.float32),
        pltpu.VMEM_SHARED((DSTN, 32), jnp.float32),
        pltpu.SemaphoreType.DMA,
        pltpu.SemaphoreType.DMA,
    ],
)
def _phase2(src_hbm, dst_hbm, ev_hbm, m_hbm, zagg_hbm, zseg_hbm,
            agg_out, seg_out, srcc, dstc, evc, mbuf,
            agg_sp, seg_sp, m_sp, sem, sem2):
    cid = lax.axis_index("c")
    sid = lax.axis_index("s")
    pltpu.sync_copy(zagg_hbm.at[pl.ds(sid * _STG, _STG)],
                    agg_sp.at[pl.ds(sid * _STG, _STG)])

    @pl.when(cid == 0)
    def _():
        pltpu.sync_copy(zseg_hbm.at[pl.ds(sid * _STG, _STG)],
                        seg_sp.at[pl.ds(sid * _STG, _STG)])

    _dstg = DSTN // 16
    pltpu.sync_copy(m_hbm.at[cid, pl.ds(DSTLO + sid * _dstg, _dstg)],
                    m_sp.at[pl.ds(sid * _dstg, _dstg)])
    plsc.subcore_barrier()
    rows_per_tile = ROWS // 16
    row0 = sid * rows_per_tile

    def chunk_body(t, carry):
        base = row0 + t * IC2
        pltpu.sync_copy(src_hbm.at[pl.ds(base, IC2)], srcc)
        pltpu.sync_copy(dst_hbm.at[pl.ds(base, IC2)], dstc)
        pltpu.sync_copy(ev_hbm.at[pl.ds(base, IC2)], evc)
        for r2 in range(IC2):
            for g in range(W // 16):
                dstc[r2, pl.ds(g * 16, 16)] = (
                    dstc[r2, pl.ds(g * 16, 16)] - DSTLO)
        for u in range(IC2 // B2):
            hs = []
            for k in range(B2):
                hs.append(pltpu.async_copy(
                    m_sp.at[dstc.at[u * B2 + k]],
                    mbuf.at[pl.ds(k * W, W)], sem))
            for h in hs:
                h.wait()

            def rcomp(r2, c):
                def gcomp(g, c2):
                    evec = evc[u * B2 + r2, pl.ds(g * 16, 16)]
                    e0 = r2 * W + g * 16
                    for j in range(16):
                        e = e0 + j
                        evs = evec[j]
                        mbuf[e, pl.ds(0, 16)] = mbuf[e, pl.ds(0, 16)] * evs
                        mbuf[e, pl.ds(16, 16)] = mbuf[e, pl.ds(16, 16)] * evs
                    return c2

                lax.fori_loop(0, W // 16, gcomp, 0)
                return c

            lax.fori_loop(0, B2, rcomp, 0)
            hs2 = []
            for k in range(B2):
                hs2.append(pltpu.async_copy(
                    mbuf.at[pl.ds(k * W, W)], agg_sp.at[srcc.at[u * B2 + k]],
                    sem2, add=True))

            @pl.when(cid == 0)
            def _():
                evhs = []
                for k in range(B2):
                    evhs.append(pltpu.async_copy(
                        evc.at[u * B2 + k], seg_sp.at[srcc.at[u * B2 + k]],
                        sem2, add=True))
                for h in evhs:
                    h.wait()

            for h in hs2:
                h.wait()
        return carry

    lax.fori_loop(0, rows_per_tile // IC2, chunk_body, 0)
    plsc.subcore_barrier()
    pltpu.sync_copy(agg_sp.at[pl.ds(sid * _STG, _STG)],
                    agg_out.at[cid, pl.ds(sid * _STG, _STG)])

    @pl.when(cid == 0)
    def _():
        pltpu.sync_copy(seg_sp.at[pl.ds(sid * _STG, _STG)],
                        seg_out.at[pl.ds(sid * _STG, _STG)])


@functools.partial(
    pl.kernel,
    out_type=jax.ShapeDtypeStruct((2, NP), jnp.float32),
    mesh=_SC_MESH,
    compiler_params=pltpu.CompilerParams(needs_layout_passes=False, use_tc_tiling_on_sc=False),
    scratch_types=[
        pltpu.VMEM((BW, W), jnp.int32),
        pltpu.VMEM((BW, W), jnp.float32),
        pltpu.VMEM_SHARED((NP,), jnp.float32),
        pltpu.SemaphoreType.DMA,
    ],
)
def _wsum(src_hbm, w_hbm, zseg_hbm, out_hbm, srcm, wm, seg_sp, sem):
    cid = lax.axis_index("c")
    sid = lax.axis_index("s")
    pltpu.sync_copy(zseg_hbm.at[pl.ds(sid * _STG, _STG)],
                    seg_sp.at[pl.ds(sid * _STG, _STG)])
    plsc.subcore_barrier()
    wid = cid * 16 + sid
    rows_per_tile = EROWS // 32
    row0 = wid * rows_per_tile

    def it_body(t, carry):
        base = row0 + t * BW
        pltpu.sync_copy(src_hbm.at[pl.ds(base, BW)], srcm)
        pltpu.sync_copy(w_hbm.at[pl.ds(base, BW)], wm)
        hs = []
        for k in range(BW):
            hs.append(pltpu.async_copy(
                wm.at[k], seg_sp.at[srcm.at[k]], sem, add=True))
        for h in hs:
            h.wait()
        return carry

    lax.fori_loop(0, rows_per_tile // BW, it_body, 0)
    plsc.subcore_barrier()
    pltpu.sync_copy(seg_sp.at[pl.ds(sid * _STG, _STG)],
                    out_hbm.at[cid, pl.ds(sid * _STG, _STG)])


def _pad_idx(n):
    return PADBASE + (jnp.arange(n, dtype=jnp.int32) % 64)


def _view(emb, er_src, er_dst, ee_src, ee_w, Wa, ba, w0, b0, Ws, bs, Wn, bn):
    x = jnp.pad(emb, ((0, NP - N), (0, 0)))
    src2d = jnp.concatenate([er_src.astype(jnp.int32),
                             _pad_idx(EP - E_ER)]).reshape(ROWS, W)
    dst2d = jnp.concatenate([er_dst.astype(jnp.int32),
                             _pad_idx(EP - E_ER)]).reshape(ROWS, W)
    esrc2d = jnp.concatenate([ee_src.astype(jnp.int32),
                              _pad_idx(EEP - E_EE)]).reshape(EROWS, W)
    ew2d = jnp.concatenate([ee_w, jnp.zeros(EEP - E_EE, jnp.float32)]
                           ).reshape(EROWS, W)
    zagg = jnp.zeros((NP, 32), jnp.float32)
    zseg = jnp.zeros((NP,), jnp.float32)
    ws3 = _wsum(esrc2d, ew2d, zseg).reshape(2, NP, 1)
    for l in range(L):
        W4 = jnp.concatenate([Wa[l][D:], Wa[l][:D], Wn[l], Ws[l]], axis=1)
        b4 = jnp.concatenate([ba[l], jnp.zeros(D, jnp.float32), bn[l], bs[l]]
                             ).reshape(1, 4 * D)
        asrc, adst, m01, slf = _dense(x, W4, b4)
        s01 = _phase1(src2d, dst2d, asrc, adst)
        w0f = w0[l].reshape(D)
        w0a4 = jnp.tile(w0f[0:32], 4).reshape(1, W)
        w0b4 = jnp.tile(w0f[32:64], 4).reshape(1, W)
        ev2d = _ev(s01.reshape(2, EP // 4, W), w0a4, w0b4,
                   b0[l].reshape(1, 1)).reshape(ROWS, W)
        agg, seg = _phase2(src2d, dst2d, ev2d, m01, zagg, zseg)
        x = _combine(slf, agg, m01, seg.reshape(NP, 1), ws3)
    return x


def kernel(er_src_H, er_dst_H, ee_src_H, ee_weight_H,
           er_src_T, er_dst_T, ee_src_T, ee_weight_T,
           embH, embT,
           WH_attn, bH_attn, wH_0, bH_0, WH_self, bH_self, WH_neigh, bH_neigh,
           WT_attn, bT_attn, wT_0, bT_0, WT_self, bT_self, WT_neigh, bT_neigh):
    xH = _view(embH, er_src_H, er_dst_H, ee_src_H, ee_weight_H,
               WH_attn, bH_attn, wH_0, bH_0, WH_self, bH_self,
               WH_neigh, bH_neigh)
    xT = _view(embT, er_src_T, er_dst_T, ee_src_T, ee_weight_T,
               WT_attn, bT_attn, wT_0, bT_0, WT_self, bT_self,
               WT_neigh, bT_neigh)
    return (xH[:NE], xH[NE:N], xT[:NE], xT[NE:N])
